# baseline (speedup 1.0000x reference)
import jax
import jax.numpy as jnp
from jax.experimental import pallas as pl
from jax.experimental.pallas import tpu as pltpu

N = 10000
D = 128
H = 128
EPS = 1e-3

BI = 400
NI = N // BI


def _layer_body(x_ref, w_ref, c_ref, a_ref, o_ref, z_ref):
    i = pl.program_id(0)

    @pl.when(i == 0)
    def _compute_z():
        z_ref[...] = jnp.dot(
            x_ref[...], w_ref[...], preferred_element_type=jnp.float32)

    o_ref[...] = jnp.maximum(
        jnp.dot(a_ref[...], z_ref[...], preferred_element_type=jnp.float32)
        + c_ref[...], 0.0)


def _gcn_layer(x, a, w, c):
    return pl.pallas_call(
        _layer_body,
        grid=(NI,),
        in_specs=[
            pl.BlockSpec((N, D), lambda i: (0, 0)),
            pl.BlockSpec((D, H), lambda i: (0, 0)),
            pl.BlockSpec((1, H), lambda i: (0, 0)),
            pl.BlockSpec((BI, N), lambda i: (i, 0)),
        ],
        out_specs=pl.BlockSpec((BI, H), lambda i: (i, 0)),
        out_shape=jax.ShapeDtypeStruct((N, H), jnp.float32),
        scratch_shapes=[pltpu.VMEM((N, H), jnp.float32)],
        compiler_params=pltpu.CompilerParams(
            dimension_semantics=("arbitrary",)),
    )(x, w, c, a)


def kernel(x, a, W1, b1, g1, beta1, mm1, mv1, W2, b2, g2, beta2, mm2, mv2):
    s1 = g1 / jnp.sqrt(mv1 + EPS)
    c1 = ((b1 - mm1) * s1 + beta1).reshape(1, H)
    s2 = g2 / jnp.sqrt(mv2 + EPS)
    c2 = ((b2 - mm2) * s2 + beta2).reshape(1, H)
    h1 = _gcn_layer(x, a, W1 * s1[None, :], c1)
    return _gcn_layer(h1, a, W2 * s2[None, :], c2)
